# baseline (speedup 1.0000x reference)
import dataclasses

import jax
import jax.numpy as jnp
from jax import lax
from jax.experimental import pallas as pl
from jax.experimental.pallas import tpu as pltpu
from jax.experimental.pallas import tpu_sc as plsc

DIM = 64
BATCH = 16384
REG_COEF = 1e-05
NROWS = 100000
NC = 2
NS = 16
LANES = 16
NW = NC * NS
BPW = BATCH // NW
CHUNK = 128
NCHUNK = BPW // CHUNK
GPC = CHUNK // LANES

NBLK = 8192
NGRID = (NROWS + NBLK - 1) // NBLK


def _norm_body(ttu_ref, tti_ref, nu_ref, ni_ref, bu_ref, bi_ref):
    u = ttu_ref[...]
    i = tti_ref[...]
    nu_ref[...] = jnp.sum(u * u, axis=0)
    ni_ref[...] = jnp.sum(i * i, axis=0)
    bu_ref[...] = u.astype(jnp.bfloat16)
    bi_ref[...] = i.astype(jnp.bfloat16)


def _row_norms(user_table, item_table):
    return pl.pallas_call(
        _norm_body,
        grid=(NGRID,),
        in_specs=[
            pl.BlockSpec((DIM, NBLK), lambda i: (0, i)),
            pl.BlockSpec((DIM, NBLK), lambda i: (0, i)),
        ],
        out_specs=[
            pl.BlockSpec((NBLK,), lambda i: (i,)),
            pl.BlockSpec((NBLK,), lambda i: (i,)),
            pl.BlockSpec((DIM, NBLK), lambda i: (0, i)),
            pl.BlockSpec((DIM, NBLK), lambda i: (0, i)),
        ],
        out_shape=[
            jax.ShapeDtypeStruct((NROWS,), jnp.float32),
            jax.ShapeDtypeStruct((NROWS,), jnp.float32),
            jax.ShapeDtypeStruct((DIM, NROWS), jnp.bfloat16),
            jax.ShapeDtypeStruct((DIM, NROWS), jnp.bfloat16),
        ],
    )(user_table.T, item_table.T)


def _sc_body(idx_u, idx_p, idx_n, hid_u, hid_p, hid_n, lid_u, lid_p, lid_n,
             utab, itab, nu2d, ni2d, diff_hbm, sq_hbm,
             iu_v, ip_v, in_v, hu_v, hp_v, hn_v, lu_v, lp_v, ln_v,
             ru0, ru1, rp0, rp1, rn0, rn1,
             mu0, mu1, mp0, mp1, mn0, mn1,
             scores_v, sq_v, sem0, sem1):
    wid = lax.axis_index("s") * NC + lax.axis_index("c")

    pltpu.sync_copy(idx_u.at[wid], iu_v)
    pltpu.sync_copy(idx_p.at[wid], ip_v)
    pltpu.sync_copy(idx_n.at[wid], in_v)
    pltpu.sync_copy(hid_u.at[wid], hu_v)
    pltpu.sync_copy(hid_p.at[wid], hp_v)
    pltpu.sync_copy(hid_n.at[wid], hn_v)
    pltpu.sync_copy(lid_u.at[wid], lu_v)
    pltpu.sync_copy(lid_p.at[wid], lp_v)
    pltpu.sync_copy(lid_n.at[wid], ln_v)

    rbufs = [(ru0, rp0, rn0, mu0, mp0, mn0), (ru1, rp1, rn1, mu1, mp1, mn1)]
    sems = [sem0, sem1]

    def fire(c):
        ru, rp, rn, mu, mp, mn = rbufs[c % 2]
        sem = sems[c % 2]
        return [
            pltpu.async_copy(utab.at[iu_v.at[c]], ru, sem),
            pltpu.async_copy(itab.at[ip_v.at[c]], rp, sem),
            pltpu.async_copy(itab.at[in_v.at[c]], rn, sem),
            pltpu.async_copy(nu2d.at[hu_v.at[c]], mu, sem),
            pltpu.async_copy(ni2d.at[hp_v.at[c]], mp, sem),
            pltpu.async_copy(ni2d.at[hn_v.at[c]], mn, sem),
        ]

    def halves(ref, r):
        a = ref[r, pl.ds(0, 32)]
        b = ref[r, pl.ds(32, 32)]
        a0, a1 = plsc.unpack(a, format=plsc.PackFormat.INTERLEAVED)
        b0, b1 = plsc.unpack(b, format=plsc.PackFormat.INTERLEAVED)
        return a0, a1, b0, b1

    sq_v[...] = jnp.zeros((LANES,), jnp.float32)
    iota = lax.iota(jnp.int32, LANES)

    pending = fire(0)
    for c in range(NCHUNK):
        nxt = fire(c + 1) if c + 1 < NCHUNK else []
        for cpd in pending:
            cpd.wait()
        pending = nxt
        ru, rp, rn, mu, mp, mn = rbufs[c % 2]

        @pl.loop(0, CHUNK)
        def _row(r):
            u0, u1, u2, u3 = halves(ru, r)
            p0, p1, p2, p3 = halves(rp, r)
            n0, n1, n2, n3 = halves(rn, r)
            s = u0 * (p0 - n0) + u1 * (p1 - n1)
            s = s + u2 * (p2 - n2) + u3 * (p3 - n3)
            scores_v[c * CHUNK + r] = s

        @pl.loop(0, GPC)
        def _norms(g):
            row = g * LANES + iota
            cu = lu_v[c, pl.ds(g * LANES, LANES)]
            cp_ = lp_v[c, pl.ds(g * LANES, LANES)]
            cn = ln_v[c, pl.ds(g * LANES, LANES)]
            nu = plsc.load_gather(mu, [row, cu])
            np_ = plsc.load_gather(mp, [row, cp_])
            nn = plsc.load_gather(mn, [row, cn])
            sq_v[...] += nu + np_ + nn

    pltpu.sync_copy(scores_v, diff_hbm.at[pl.ds(wid * BPW, BPW)])
    pltpu.sync_copy(sq_v, sq_hbm.at[wid])


def _loss_body(diff_ref, sq_ref, out_ref):
    ones = jnp.ones((LANES, 1), jnp.float32)
    d = lax.dot_general(diff_ref[...], ones, (((1,), (0,)), ((), ())),
                        preferred_element_type=jnp.float32)[:, 0]
    sp = jnp.maximum(-d, 0.0) + jnp.log1p(jnp.exp(-jnp.abs(d)))
    out_ref[0] = jnp.sum(sp) * (1.0 / BATCH)
    out_ref[1] = (0.5 * REG_COEF) * jnp.sum(sq_ref[...])


@jax.jit
def kernel(userids, itemids_pos, itemids_neg, user_table, item_table):
    shp = (NW, NCHUNK, CHUNK)
    ids = [x.astype(jnp.int32) for x in (userids, itemids_pos, itemids_neg)]
    gidx = [x.reshape(shp) for x in ids]
    hidx = [(x >> 4).reshape(shp) for x in ids]
    lidx = [(x & 15).reshape(shp) for x in ids]

    norms_u, norms_i, ubf_t, ibf_t = _row_norms(user_table, item_table)
    utab_bf = ubf_t.T
    itab_bf = ibf_t.T
    nu2d = norms_u.reshape(NROWS // LANES, LANES)
    ni2d = norms_i.reshape(NROWS // LANES, LANES)

    mesh = plsc.VectorSubcoreMesh(
        core_axis_name="c", subcore_axis_name="s",
        num_cores=NC, num_subcores=NS)

    cp = pltpu.CompilerParams()
    if "needs_layout_passes" in pltpu.CompilerParams.__dataclass_fields__:
        cp = dataclasses.replace(cp, needs_layout_passes=False)
    if "use_tc_tiling_on_sc" in pltpu.CompilerParams.__dataclass_fields__:
        cp = dataclasses.replace(cp, use_tc_tiling_on_sc=False)

    idx_t = pltpu.VMEM((NCHUNK, CHUNK), jnp.int32)
    row_t = pltpu.VMEM((CHUNK, DIM), jnp.bfloat16)
    nrm_t = pltpu.VMEM((CHUNK, LANES), jnp.float32)
    sc = pl.kernel(
        _sc_body,
        compiler_params=cp,
        out_type=[
            jax.ShapeDtypeStruct((BATCH, LANES), jnp.float32),
            jax.ShapeDtypeStruct((NW, LANES), jnp.float32),
        ],
        mesh=mesh,
        scratch_types=[
            idx_t, idx_t, idx_t, idx_t, idx_t, idx_t, idx_t, idx_t, idx_t,
            row_t, row_t, row_t, row_t, row_t, row_t,
            nrm_t, nrm_t, nrm_t, nrm_t, nrm_t, nrm_t,
            pltpu.VMEM((BPW, LANES), jnp.float32),
            pltpu.VMEM((LANES,), jnp.float32),
            pltpu.SemaphoreType.DMA,
            pltpu.SemaphoreType.DMA,
        ],
    )
    diff, sq = sc(*gidx, *hidx, *lidx, utab_bf, itab_bf, nu2d, ni2d)

    out = pl.pallas_call(
        _loss_body,
        out_shape=jax.ShapeDtypeStruct((2,), jnp.float32),
        out_specs=pl.BlockSpec(memory_space=pltpu.SMEM),
    )(diff, sq)
    return out[0], out[1]


# speedup vs baseline: 1.3161x; 1.3161x over previous
import dataclasses

import jax
import jax.numpy as jnp
from jax import lax
from jax.experimental import pallas as pl
from jax.experimental.pallas import tpu as pltpu
from jax.experimental.pallas import tpu_sc as plsc

DIM = 64
BATCH = 16384
REG_COEF = 1e-05
NC = 2
NS = 16
LANES = 16
NW = NC * NS
BPW = BATCH // NW
CHUNK = 128
NCHUNK = BPW // CHUNK
GPC = CHUNK // LANES


def _sc_body(idx_u, idx_p, idx_n, utab, itab, diff_hbm, sq_hbm,
             iu_v, ip_v, in_v,
             ru0, ru1, rp0, rp1, rn0, rn1,
             scores_v, sq_v, sem0, sem1):
    wid = lax.axis_index("s") * NC + lax.axis_index("c")

    pltpu.sync_copy(idx_u.at[wid], iu_v)
    pltpu.sync_copy(idx_p.at[wid], ip_v)
    pltpu.sync_copy(idx_n.at[wid], in_v)

    rbufs = [(ru0, rp0, rn0), (ru1, rp1, rn1)]
    sems = [sem0, sem1]

    def fire(c):
        ru, rp, rn = rbufs[c % 2]
        sem = sems[c % 2]
        return [
            pltpu.async_copy(utab.at[iu_v.at[c]], ru, sem),
            pltpu.async_copy(itab.at[ip_v.at[c]], rp, sem),
            pltpu.async_copy(itab.at[in_v.at[c]], rn, sem),
        ]

    sq_v[...] = jnp.zeros((LANES,), jnp.float32)
    iota = lax.iota(jnp.int32, LANES)

    pending = fire(0)
    for c in range(NCHUNK):
        nxt = fire(c + 1) if c + 1 < NCHUNK else []
        for cpd in pending:
            cpd.wait()
        pending = nxt
        ru, rp, rn = rbufs[c % 2]

        @pl.loop(0, GPC)
        def _group(g):
            row = g * LANES + iota
            pos = jnp.zeros((LANES,), jnp.float32)
            neg = jnp.zeros((LANES,), jnp.float32)
            sq = jnp.zeros((LANES,), jnp.float32)
            for d in range(DIM):
                col = jnp.full((LANES,), d, jnp.int32)
                u = plsc.load_gather(ru, [row, col])
                p = plsc.load_gather(rp, [row, col])
                n = plsc.load_gather(rn, [row, col])
                pos = pos + u * p
                neg = neg + u * n
                sq = sq + (u * u + p * p + n * n)
            scores_v[pl.ds(c * CHUNK + g * LANES, LANES)] = pos - neg
            sq_v[...] += sq

    pltpu.sync_copy(scores_v, diff_hbm.at[pl.ds(wid * BPW, BPW)])
    pltpu.sync_copy(sq_v, sq_hbm.at[wid])


def _loss_body(diff_ref, sq_ref, out_ref):
    d = diff_ref[...]
    sp = jnp.maximum(-d, 0.0) + jnp.log1p(jnp.exp(-jnp.abs(d)))
    out_ref[0] = jnp.sum(sp) * (1.0 / BATCH)
    out_ref[1] = (0.5 * REG_COEF) * jnp.sum(sq_ref[...])


@jax.jit
def kernel(userids, itemids_pos, itemids_neg, user_table, item_table):
    shp = (NW, NCHUNK, CHUNK)
    gidx = [x.astype(jnp.int32).reshape(shp)
            for x in (userids, itemids_pos, itemids_neg)]

    mesh = plsc.VectorSubcoreMesh(
        core_axis_name="c", subcore_axis_name="s",
        num_cores=NC, num_subcores=NS)

    cp = pltpu.CompilerParams()
    if "needs_layout_passes" in pltpu.CompilerParams.__dataclass_fields__:
        cp = dataclasses.replace(cp, needs_layout_passes=False)
    if "use_tc_tiling_on_sc" in pltpu.CompilerParams.__dataclass_fields__:
        cp = dataclasses.replace(cp, use_tc_tiling_on_sc=False)

    idx_t = pltpu.VMEM((NCHUNK, CHUNK), jnp.int32)
    row_t = pltpu.VMEM((CHUNK, DIM), jnp.float32)
    sc = pl.kernel(
        _sc_body,
        compiler_params=cp,
        out_type=[
            jax.ShapeDtypeStruct((BATCH,), jnp.float32),
            jax.ShapeDtypeStruct((NW, LANES), jnp.float32),
        ],
        mesh=mesh,
        scratch_types=[
            idx_t, idx_t, idx_t,
            row_t, row_t, row_t, row_t, row_t, row_t,
            pltpu.VMEM((BPW,), jnp.float32),
            pltpu.VMEM((LANES,), jnp.float32),
            pltpu.SemaphoreType.DMA,
            pltpu.SemaphoreType.DMA,
        ],
    )
    diff, sq = sc(*gidx, user_table, item_table)

    out = pl.pallas_call(
        _loss_body,
        out_shape=jax.ShapeDtypeStruct((2,), jnp.float32),
        out_specs=pl.BlockSpec(memory_space=pltpu.SMEM),
    )(diff.reshape(BATCH // 128, 128), sq)
    return out[0], out[1]
